# Initial kernel scaffold; baseline (speedup 1.0000x reference)
import functools
import math

import jax
import jax.numpy as jnp
from jax.experimental import pallas as pl
from jax.experimental.pallas import tpu as pltpu

N = 10000
E = 160000
D = 128
NB = 128
H = 4
DH = D // H
L = 6
NG = 512
MAX_RADIUS = 5.0
AVG_DEGREE = 15.57930850982666
AVG_NUM_NODES = 18.03065905448718

NP_ = 10240
BN = 1024
BE = 8000


def _ln(x):
    m = jnp.mean(x, axis=-1, keepdims=True)
    v = jnp.mean((x - m) ** 2, axis=-1, keepdims=True)
    return (x - m) * jax.lax.rsqrt(v + 1e-5)


def _rbf_from_len(elen):
    width = MAX_RADIUS / NB
    centers = jax.lax.broadcasted_iota(jnp.float32, (1, NB), 1) * (
        MAX_RADIUS / (NB - 1))
    z = (elen - centers) * (1.0 / width)
    return jnp.exp(-(z * z))


def _gate_block(geom, wg_rbf, wg_sh):
    elen = geom[:, 0:1]
    ux = geom[:, 1:2]
    uy = geom[:, 2:3]
    uz = geom[:, 3:4]
    rbf = _rbf_from_len(elen)
    gate = jnp.dot(rbf, wg_rbf, preferred_element_type=jnp.float32)
    s3 = math.sqrt(3.0)
    s15 = math.sqrt(15.0)
    s5 = math.sqrt(5.0)
    coefs = [
        jnp.ones_like(ux),
        s3 * ux, s3 * uy, s3 * uz,
        s15 * ux * uy, s15 * uy * uz,
        0.5 * s5 * (3.0 * uz * uz - 1.0),
        s15 * ux * uz, 0.5 * s15 * (ux * ux - uy * uy),
    ]
    for j, c in enumerate(coefs):
        gate = gate + c * wg_sh[j:j + 1, :]
    return gate


def _qkv_body(x_ref, wq_ref, wk_ref, wv_ref, q_ref, k_ref, v_ref):
    h = _ln(x_ref[...])
    q_ref[...] = jnp.dot(h, wq_ref[...], preferred_element_type=jnp.float32)
    k_ref[...] = jnp.dot(h, wk_ref[...], preferred_element_type=jnp.float32)
    v_ref[...] = jnp.dot(h, wv_ref[...], preferred_element_type=jnp.float32)


def _qkv(x, wq, wk, wv):
    grid = (NP_ // BN,)
    bspec_x = pl.BlockSpec((BN, D), lambda i: (i, 0))
    bspec_w = pl.BlockSpec((D, D), lambda i: (0, 0))
    out = pl.pallas_call(
        _qkv_body,
        grid=grid,
        in_specs=[bspec_x, bspec_w, bspec_w, bspec_w],
        out_specs=[bspec_x, bspec_x, bspec_x],
        out_shape=[jax.ShapeDtypeStruct((NP_, D), jnp.float32)] * 3,
    )(x, wq, wk, wv)
    return out


def _alpha_body(geom_ref, ks_ref, qd_ref, wgr_ref, wgs_ref, mh_ref, a_ref):
    gate = _gate_block(geom_ref[...], wgr_ref[...], wgs_ref[...])
    p = qd_ref[...] * ks_ref[...] * gate
    a_ref[...] = jnp.dot(p, mh_ref[...], preferred_element_type=jnp.float32)


def _alpha(geom, ks, qd, wgr, wgs, mh):
    grid = (E // BE,)
    be = pl.BlockSpec((BE, 4), lambda i: (i, 0))
    bd = pl.BlockSpec((BE, D), lambda i: (i, 0))
    bw = pl.BlockSpec((D, D), lambda i: (0, 0))
    bs = pl.BlockSpec((16, D), lambda i: (0, 0))
    bm = pl.BlockSpec((D, 8), lambda i: (0, 0))
    bo = pl.BlockSpec((BE, 8), lambda i: (i, 0))
    return pl.pallas_call(
        _alpha_body,
        grid=grid,
        in_specs=[be, bd, bd, bw, bs, bm],
        out_specs=bo,
        out_shape=jax.ShapeDtypeStruct((E, 8), jnp.float32),
    )(geom, ks, qd, wgr, wgs, mh)


def _msg_body(ex_ref, dn_ref, vs_ref, me_ref, o_ref):
    a = ex_ref[...] / (dn_ref[...] + 1e-9)
    o_ref[...] = vs_ref[...] * jnp.dot(
        a, me_ref[...], preferred_element_type=jnp.float32)


def _msg(ex, dn, vs, me):
    grid = (E // BE,)
    b8 = pl.BlockSpec((BE, 8), lambda i: (i, 0))
    bd = pl.BlockSpec((BE, D), lambda i: (i, 0))
    bm = pl.BlockSpec((8, D), lambda i: (0, 0))
    return pl.pallas_call(
        _msg_body,
        grid=grid,
        in_specs=[b8, b8, bd, bm],
        out_specs=bd,
        out_shape=jax.ShapeDtypeStruct((E, D), jnp.float32),
    )(ex, dn, vs, me)


def _tail_body(x_ref, agg_ref, wo_ref, w1_ref, w2_ref, o_ref):
    t = x_ref[...] + jnp.dot(agg_ref[...], wo_ref[...],
                             preferred_element_type=jnp.float32)
    h2 = _ln(t)
    u = jnp.dot(h2, w1_ref[...], preferred_element_type=jnp.float32)
    u = u * (1.0 / (1.0 + jnp.exp(-u)))
    o_ref[...] = t + jnp.dot(u, w2_ref[...], preferred_element_type=jnp.float32)


def _tail(x, agg, wo, w1, w2):
    grid = (NP_ // BN,)
    bx = pl.BlockSpec((BN, D), lambda i: (i, 0))
    bwo = pl.BlockSpec((D, D), lambda i: (0, 0))
    bw1 = pl.BlockSpec((D, 2 * D), lambda i: (0, 0))
    bw2 = pl.BlockSpec((2 * D, D), lambda i: (0, 0))
    return pl.pallas_call(
        _tail_body,
        grid=grid,
        in_specs=[bx, bx, bwo, bw1, bw2],
        out_specs=bx,
        out_shape=jax.ShapeDtypeStruct((NP_, D), jnp.float32),
    )(x, agg, wo, w1, w2)


def _deg_body(geom_ref, wdeg_ref, o_ref):
    rbf = _rbf_from_len(geom_ref[:, 0:1])
    o_ref[...] = jnp.dot(rbf, wdeg_ref[...], preferred_element_type=jnp.float32)


def _deg(geom, wdeg):
    grid = (E // BE,)
    be = pl.BlockSpec((BE, 4), lambda i: (i, 0))
    bw = pl.BlockSpec((D, D), lambda i: (0, 0))
    bd = pl.BlockSpec((BE, D), lambda i: (i, 0))
    return pl.pallas_call(
        _deg_body,
        grid=grid,
        in_specs=[be, bw],
        out_specs=bd,
        out_shape=jax.ShapeDtypeStruct((E, D), jnp.float32),
    )(geom, wdeg)


def _head_body(x_ref, b_ref, w1_ref, w2_ref, o_ref):
    i = pl.program_id(0)

    @pl.when(i == 0)
    def _():
        o_ref[...] = jnp.zeros_like(o_ref)

    h = _ln(x_ref[...])
    u = jnp.dot(h, w1_ref[...], preferred_element_type=jnp.float32)
    u = u * (1.0 / (1.0 + jnp.exp(-u)))
    ne = jnp.dot(u, w2_ref[...], preferred_element_type=jnp.float32)
    ne0 = ne[:, 0:1]
    gids = jax.lax.broadcasted_iota(jnp.int32, (1, NG), 1)
    onehot = (b_ref[...] == gids).astype(jnp.float32)
    o_ref[...] += jnp.dot(ne0.T, onehot, preferred_element_type=jnp.float32)


def _head(x, batch2d, w1, w2p):
    grid = (NP_ // BN,)
    bx = pl.BlockSpec((BN, D), lambda i: (i, 0))
    bb = pl.BlockSpec((BN, 1), lambda i: (i, 0))
    bw1 = pl.BlockSpec((D, D), lambda i: (0, 0))
    bw2 = pl.BlockSpec((D, 8), lambda i: (0, 0))
    bo = pl.BlockSpec((1, NG), lambda i: (0, 0))
    return pl.pallas_call(
        _head_body,
        grid=grid,
        in_specs=[bx, bb, bw1, bw2],
        out_specs=bo,
        out_shape=jax.ShapeDtypeStruct((1, NG), jnp.float32),
    )(x, batch2d, w1, w2p)


def kernel(node_atom, pos, batch, edge_index, emb_table, W_deg, Wq, Wk, Wv,
           Wo, Wg_rbf, Wg_sh, W1, W2, head_w1, head_w2):
    src = edge_index[0]
    dst = edge_index[1]

    pvec = pos[src] - pos[dst]
    elen = jnp.sqrt(jnp.sum(pvec ** 2, axis=1) + 1e-12)
    unit = pvec / elen[:, None]
    geom = jnp.concatenate([elen[:, None], unit], axis=1)

    hm = (jax.lax.broadcasted_iota(jnp.int32, (D, 8), 0) // DH ==
          jax.lax.broadcasted_iota(jnp.int32, (D, 8), 1))
    mh = hm.astype(jnp.float32) * (1.0 / math.sqrt(float(DH)))
    me = (jax.lax.broadcasted_iota(jnp.int32, (8, D), 1) // DH ==
          jax.lax.broadcasted_iota(jnp.int32, (8, D), 0)).astype(jnp.float32)

    wgs_pad = jnp.zeros((L, 16, D), jnp.float32).at[:, :9, :].set(Wg_sh)

    z = _deg(geom, W_deg)
    deg = jax.ops.segment_sum(z, dst, num_segments=N) / AVG_DEGREE
    x0 = emb_table[node_atom] + deg
    x = jnp.zeros((NP_, D), jnp.float32).at[:N].set(x0)

    for l in range(L):
        qn, kn, vn = _qkv(x, Wq[l], Wk[l], Wv[l])
        qd = jnp.take(qn, dst, axis=0)
        ks = jnp.take(kn, src, axis=0)
        vs = jnp.take(vn, src, axis=0)
        alpha = _alpha(geom, ks, qd, Wg_rbf[l], wgs_pad[l], mh)
        m = jax.ops.segment_max(alpha, dst, num_segments=N)
        m = jnp.where(jnp.isfinite(m), m, 0.0)
        ex = jnp.exp(alpha - m[dst])
        denom = jax.ops.segment_sum(ex, dst, num_segments=N)
        msg = _msg(ex, denom[dst], vs, me)
        agg = jax.ops.segment_sum(msg, dst, num_segments=N)
        aggp = jnp.zeros((NP_, D), jnp.float32).at[:N].set(agg)
        x = _tail(x, aggp, Wo[l], W1[l], W2[l])

    batch2d = jnp.zeros((NP_, 1), jnp.int32).at[:N, 0].set(batch.astype(jnp.int32))
    w2p = jnp.zeros((D, 8), jnp.float32).at[:, 0:1].set(head_w2)
    energy = _head(x, batch2d, head_w1, w2p)
    return energy.reshape(NG) / AVG_NUM_NODES


# speedup vs baseline: 1.0046x; 1.0046x over previous
import functools
import math

import jax
import jax.numpy as jnp
from jax.experimental import pallas as pl
from jax.experimental.pallas import tpu as pltpu

N = 10000
E = 160000
D = 128
NB = 128
H = 4
DH = D // H
L = 6
NG = 512
MAX_RADIUS = 5.0
AVG_DEGREE = 15.57930850982666
AVG_NUM_NODES = 18.03065905448718

NP_ = 10240
BN = 1024
BE = 4000


def _ln(x):
    m = jnp.mean(x, axis=-1, keepdims=True)
    v = jnp.mean((x - m) ** 2, axis=-1, keepdims=True)
    return (x - m) * jax.lax.rsqrt(v + 1e-5)


def _rbf_from_len(elen):
    width = MAX_RADIUS / NB
    centers = jax.lax.broadcasted_iota(jnp.int32, (1, NB), 1).astype(
        jnp.float32) * (MAX_RADIUS / (NB - 1))
    z = (elen - centers) * (1.0 / width)
    return jnp.exp(-(z * z))


def _gate_block(geom, wg_rbf, wg_sh):
    elen = geom[:, 0:1]
    ux = geom[:, 1:2]
    uy = geom[:, 2:3]
    uz = geom[:, 3:4]
    rbf = _rbf_from_len(elen)
    gate = jnp.dot(rbf, wg_rbf, preferred_element_type=jnp.float32)
    s3 = math.sqrt(3.0)
    s15 = math.sqrt(15.0)
    s5 = math.sqrt(5.0)
    coefs = [
        jnp.ones_like(ux),
        s3 * ux, s3 * uy, s3 * uz,
        s15 * ux * uy, s15 * uy * uz,
        0.5 * s5 * (3.0 * uz * uz - 1.0),
        s15 * ux * uz, 0.5 * s15 * (ux * ux - uy * uy),
    ]
    for j, c in enumerate(coefs):
        gate = gate + c * wg_sh[j:j + 1, :]
    return gate


def _qkv_body(x_ref, wq_ref, wk_ref, wv_ref, q_ref, k_ref, v_ref):
    h = _ln(x_ref[...])
    q_ref[...] = jnp.dot(h, wq_ref[...], preferred_element_type=jnp.float32)
    k_ref[...] = jnp.dot(h, wk_ref[...], preferred_element_type=jnp.float32)
    v_ref[...] = jnp.dot(h, wv_ref[...], preferred_element_type=jnp.float32)


def _qkv(x, wq, wk, wv):
    grid = (NP_ // BN,)
    bspec_x = pl.BlockSpec((BN, D), lambda i: (i, 0))
    bspec_w = pl.BlockSpec((D, D), lambda i: (0, 0))
    out = pl.pallas_call(
        _qkv_body,
        grid=grid,
        in_specs=[bspec_x, bspec_w, bspec_w, bspec_w],
        out_specs=[bspec_x, bspec_x, bspec_x],
        out_shape=[jax.ShapeDtypeStruct((NP_, D), jnp.float32)] * 3,
    )(x, wq, wk, wv)
    return out


def _alpha_body(geom_ref, ks_ref, qd_ref, wgr_ref, wgs_ref, mh_ref, a_ref):
    gate = _gate_block(geom_ref[...], wgr_ref[...], wgs_ref[...])
    p = qd_ref[...] * ks_ref[...] * gate
    a_ref[...] = jnp.dot(p, mh_ref[...], preferred_element_type=jnp.float32)


def _alpha(geom, ks, qd, wgr, wgs, mh):
    grid = (E // BE,)
    be = pl.BlockSpec((BE, 4), lambda i: (i, 0))
    bd = pl.BlockSpec((BE, D), lambda i: (i, 0))
    bw = pl.BlockSpec((D, D), lambda i: (0, 0))
    bs = pl.BlockSpec((16, D), lambda i: (0, 0))
    bm = pl.BlockSpec((D, 8), lambda i: (0, 0))
    bo = pl.BlockSpec((BE, 8), lambda i: (i, 0))
    return pl.pallas_call(
        _alpha_body,
        grid=grid,
        in_specs=[be, bd, bd, bw, bs, bm],
        out_specs=bo,
        out_shape=jax.ShapeDtypeStruct((E, 8), jnp.float32),
    )(geom, ks, qd, wgr, wgs, mh)


def _msg_body(ex_ref, dn_ref, vs_ref, me_ref, o_ref):
    a = ex_ref[...] / (dn_ref[...] + 1e-9)
    o_ref[...] = vs_ref[...] * jnp.dot(
        a, me_ref[...], preferred_element_type=jnp.float32)


def _msg(ex, dn, vs, me):
    grid = (E // BE,)
    b8 = pl.BlockSpec((BE, 8), lambda i: (i, 0))
    bd = pl.BlockSpec((BE, D), lambda i: (i, 0))
    bm = pl.BlockSpec((8, D), lambda i: (0, 0))
    return pl.pallas_call(
        _msg_body,
        grid=grid,
        in_specs=[b8, b8, bd, bm],
        out_specs=bd,
        out_shape=jax.ShapeDtypeStruct((E, D), jnp.float32),
    )(ex, dn, vs, me)


def _tail_body(x_ref, agg_ref, wo_ref, w1_ref, w2_ref, o_ref):
    t = x_ref[...] + jnp.dot(agg_ref[...], wo_ref[...],
                             preferred_element_type=jnp.float32)
    h2 = _ln(t)
    u = jnp.dot(h2, w1_ref[...], preferred_element_type=jnp.float32)
    u = u * (1.0 / (1.0 + jnp.exp(-u)))
    o_ref[...] = t + jnp.dot(u, w2_ref[...], preferred_element_type=jnp.float32)


def _tail(x, agg, wo, w1, w2):
    grid = (NP_ // BN,)
    bx = pl.BlockSpec((BN, D), lambda i: (i, 0))
    bwo = pl.BlockSpec((D, D), lambda i: (0, 0))
    bw1 = pl.BlockSpec((D, 2 * D), lambda i: (0, 0))
    bw2 = pl.BlockSpec((2 * D, D), lambda i: (0, 0))
    return pl.pallas_call(
        _tail_body,
        grid=grid,
        in_specs=[bx, bx, bwo, bw1, bw2],
        out_specs=bx,
        out_shape=jax.ShapeDtypeStruct((NP_, D), jnp.float32),
    )(x, agg, wo, w1, w2)


def _deg_body(geom_ref, wdeg_ref, o_ref):
    rbf = _rbf_from_len(geom_ref[:, 0:1])
    o_ref[...] = jnp.dot(rbf, wdeg_ref[...], preferred_element_type=jnp.float32)


def _deg(geom, wdeg):
    grid = (E // BE,)
    be = pl.BlockSpec((BE, 4), lambda i: (i, 0))
    bw = pl.BlockSpec((D, D), lambda i: (0, 0))
    bd = pl.BlockSpec((BE, D), lambda i: (i, 0))
    return pl.pallas_call(
        _deg_body,
        grid=grid,
        in_specs=[be, bw],
        out_specs=bd,
        out_shape=jax.ShapeDtypeStruct((E, D), jnp.float32),
    )(geom, wdeg)


def _head_body(x_ref, b_ref, w1_ref, w2_ref, o_ref):
    i = pl.program_id(0)

    @pl.when(i == 0)
    def _():
        o_ref[...] = jnp.zeros_like(o_ref)

    h = _ln(x_ref[...])
    u = jnp.dot(h, w1_ref[...], preferred_element_type=jnp.float32)
    u = u * (1.0 / (1.0 + jnp.exp(-u)))
    ne = jnp.dot(u, w2_ref[...], preferred_element_type=jnp.float32)
    ne0 = ne[:, 0:1]
    gids = jax.lax.broadcasted_iota(jnp.int32, (1, NG), 1)
    onehot = (b_ref[...] == gids).astype(jnp.float32)
    o_ref[...] += jnp.dot(ne0.T, onehot, preferred_element_type=jnp.float32)


def _head(x, batch2d, w1, w2p):
    grid = (NP_ // BN,)
    bx = pl.BlockSpec((BN, D), lambda i: (i, 0))
    bb = pl.BlockSpec((BN, 1), lambda i: (i, 0))
    bw1 = pl.BlockSpec((D, D), lambda i: (0, 0))
    bw2 = pl.BlockSpec((D, 8), lambda i: (0, 0))
    bo = pl.BlockSpec((1, NG), lambda i: (0, 0))
    return pl.pallas_call(
        _head_body,
        grid=grid,
        in_specs=[bx, bb, bw1, bw2],
        out_specs=bo,
        out_shape=jax.ShapeDtypeStruct((1, NG), jnp.float32),
    )(x, batch2d, w1, w2p)


def kernel(node_atom, pos, batch, edge_index, emb_table, W_deg, Wq, Wk, Wv,
           Wo, Wg_rbf, Wg_sh, W1, W2, head_w1, head_w2):
    src = edge_index[0]
    dst = edge_index[1]

    pvec = pos[src] - pos[dst]
    elen = jnp.sqrt(jnp.sum(pvec ** 2, axis=1) + 1e-12)
    unit = pvec / elen[:, None]
    geom = jnp.concatenate([elen[:, None], unit], axis=1)

    hm = (jax.lax.broadcasted_iota(jnp.int32, (D, 8), 0) // DH ==
          jax.lax.broadcasted_iota(jnp.int32, (D, 8), 1))
    mh = hm.astype(jnp.float32) * (1.0 / math.sqrt(float(DH)))
    me = (jax.lax.broadcasted_iota(jnp.int32, (8, D), 1) // DH ==
          jax.lax.broadcasted_iota(jnp.int32, (8, D), 0)).astype(jnp.float32)

    wgs_pad = jnp.zeros((L, 16, D), jnp.float32).at[:, :9, :].set(Wg_sh)

    z = _deg(geom, W_deg)
    deg = jax.ops.segment_sum(z, dst, num_segments=N) / AVG_DEGREE
    x0 = emb_table[node_atom] + deg
    x = jnp.zeros((NP_, D), jnp.float32).at[:N].set(x0)

    for l in range(L):
        qn, kn, vn = _qkv(x, Wq[l], Wk[l], Wv[l])
        qd = jnp.take(qn, dst, axis=0)
        ks = jnp.take(kn, src, axis=0)
        vs = jnp.take(vn, src, axis=0)
        alpha = _alpha(geom, ks, qd, Wg_rbf[l], wgs_pad[l], mh)
        m = jax.ops.segment_max(alpha, dst, num_segments=N)
        m = jnp.where(jnp.isfinite(m), m, 0.0)
        ex = jnp.exp(alpha - m[dst])
        denom = jax.ops.segment_sum(ex, dst, num_segments=N)
        msg = _msg(ex, denom[dst], vs, me)
        agg = jax.ops.segment_sum(msg, dst, num_segments=N)
        aggp = jnp.zeros((NP_, D), jnp.float32).at[:N].set(agg)
        x = _tail(x, aggp, Wo[l], W1[l], W2[l])

    batch2d = jnp.zeros((NP_, 1), jnp.int32).at[:N, 0].set(batch.astype(jnp.int32))
    w2p = jnp.zeros((D, 8), jnp.float32).at[:, 0:1].set(head_w2)
    energy = _head(x, batch2d, head_w1, w2p)
    return energy.reshape(NG) / AVG_NUM_NODES


# speedup vs baseline: 2.0541x; 2.0448x over previous
import functools
import math

import jax
import jax.numpy as jnp
from jax import lax
from jax.experimental import pallas as pl
from jax.experimental.pallas import tpu as pltpu
from jax.experimental.pallas import tpu_sc as plsc

N = 10000
E = 160000
D = 128
NB = 128
H = 4
DH = D // H
L = 6
NG = 512
MAX_RADIUS = 5.0
AVG_DEGREE = 15.57930850982666
AVG_NUM_NODES = 18.03065905448718

NP_ = 10240
BN = 1024
EP = 163840
BE = 4096
NWK = 32
EPW = EP // NWK
CH = 128
NCH = EPW // CH
NROW = NP_ // 16
NRC = NROW // CH


def _ln(x):
    m = jnp.mean(x, axis=-1, keepdims=True)
    v = jnp.mean((x - m) ** 2, axis=-1, keepdims=True)
    return (x - m) * jax.lax.rsqrt(v + 1e-5)


def _rbf_from_len(elen):
    width = MAX_RADIUS / NB
    centers = jax.lax.broadcasted_iota(jnp.int32, (1, NB), 1).astype(
        jnp.float32) * (MAX_RADIUS / (NB - 1))
    z = (elen - centers) * (1.0 / width)
    return jnp.exp(-(z * z))


def _gate_block(geom, wg_rbf, wg_sh):
    elen = geom[:, 0:1]
    ux = geom[:, 1:2]
    uy = geom[:, 2:3]
    uz = geom[:, 3:4]
    rbf = _rbf_from_len(elen)
    gate = jnp.dot(rbf, wg_rbf, preferred_element_type=jnp.float32)
    s3 = math.sqrt(3.0)
    s15 = math.sqrt(15.0)
    s5 = math.sqrt(5.0)
    coefs = [
        jnp.ones_like(ux),
        s3 * ux, s3 * uy, s3 * uz,
        s15 * ux * uy, s15 * uy * uz,
        0.5 * s5 * (3.0 * uz * uz - 1.0),
        s15 * ux * uz, 0.5 * s15 * (ux * ux - uy * uy),
    ]
    for j, c in enumerate(coefs):
        gate = gate + c * wg_sh[j:j + 1, :]
    return gate


_MESH = plsc.VectorSubcoreMesh(core_axis_name="c", subcore_axis_name="s")


@functools.partial(
    pl.kernel,
    mesh=_MESH,
    out_type=[
        jax.ShapeDtypeStruct((EP, D), jnp.float32),
        jax.ShapeDtypeStruct((EP, D), jnp.float32),
        jax.ShapeDtypeStruct((EP, D), jnp.float32),
    ],
    scratch_types=[
        pltpu.VMEM((CH,), jnp.int32),
        pltpu.VMEM((CH,), jnp.int32),
        pltpu.VMEM((CH, D), jnp.float32),
        pltpu.SemaphoreType.DMA,
    ],
)
def _sc_gather3(qn, kn, vn, dst, src, qd, kd, vd, idxd_v, idxs_v, rows_v, sem):
    cid = lax.axis_index("c")
    sid = lax.axis_index("s")
    base0 = (cid * 16 + sid) * EPW

    def chunk(it, _):
        base = base0 + it * CH
        pltpu.sync_copy(dst.at[pl.ds(base, CH)], idxd_v)
        pltpu.sync_copy(src.at[pl.ds(base, CH)], idxs_v)
        pltpu.async_copy(qn.at[idxd_v], rows_v, sem).wait()
        pltpu.sync_copy(rows_v, qd.at[pl.ds(base, CH)])
        pltpu.async_copy(kn.at[idxs_v], rows_v, sem).wait()
        pltpu.sync_copy(rows_v, kd.at[pl.ds(base, CH)])
        pltpu.async_copy(vn.at[idxs_v], rows_v, sem).wait()
        pltpu.sync_copy(rows_v, vd.at[pl.ds(base, CH)])
        return 0

    lax.fori_loop(0, NCH, chunk, 0)


@functools.partial(
    pl.kernel,
    mesh=_MESH,
    out_type=jax.ShapeDtypeStruct((2, NP_, D), jnp.float32),
    scratch_types=[
        pltpu.VMEM((CH,), jnp.int32),
        pltpu.VMEM((CH, D), jnp.float32),
        pltpu.VMEM_SHARED((NP_, D), jnp.float32),
    ],
)
def _sc_scatter(rows, dst, zrow, om, idx_v, rows_v, accm):
    cid = lax.axis_index("c")
    sid = lax.axis_index("s")
    rbase = sid * NROW

    def zinit(i, _):
        rb = rbase + i * CH
        pltpu.sync_copy(zrow.at[pl.ds(rb, CH)], rows_v)
        pltpu.sync_copy(rows_v, accm.at[pl.ds(rb, CH)])
        return 0

    lax.fori_loop(0, NRC, zinit, 0)
    plsc.subcore_barrier()
    base0 = (cid * 16 + sid) * EPW

    def chunk(it, _):
        base = base0 + it * CH
        pltpu.sync_copy(dst.at[pl.ds(base, CH)], idx_v)
        pltpu.sync_copy(rows.at[pl.ds(base, CH)], rows_v)
        pltpu.sync_copy(rows_v, accm.at[idx_v], add=True)
        return 0

    lax.fori_loop(0, NCH, chunk, 0)
    plsc.subcore_barrier()

    def wback(i, _):
        rb = rbase + i * CH
        pltpu.sync_copy(accm.at[pl.ds(rb, CH)], rows_v)
        pltpu.sync_copy(rows_v, om.at[cid, pl.ds(rb, CH)])
        return 0

    lax.fori_loop(0, NRC, wback, 0)


def _qkv_body(x_ref, wq_ref, wk_ref, wv_ref, q_ref, k_ref, v_ref):
    h = _ln(x_ref[...])
    q_ref[...] = jnp.dot(h, wq_ref[...], preferred_element_type=jnp.float32)
    k_ref[...] = jnp.dot(h, wk_ref[...], preferred_element_type=jnp.float32)
    v_ref[...] = jnp.dot(h, wv_ref[...], preferred_element_type=jnp.float32)


def _qkv(x, wq, wk, wv):
    bx = pl.BlockSpec((BN, D), lambda i: (i, 0))
    bw = pl.BlockSpec((D, D), lambda i: (0, 0))
    return pl.pallas_call(
        _qkv_body,
        grid=(NP_ // BN,),
        in_specs=[bx, bw, bw, bw],
        out_specs=[bx, bx, bx],
        out_shape=[jax.ShapeDtypeStruct((NP_, D), jnp.float32)] * 3,
    )(x, wq, wk, wv)


def _alpha_body(geom_ref, qd_ref, kd_ref, vd_ref, wgr_ref, wgs_ref, mh_ref,
                me_ref, msg_ref, exe_ref):
    gate = _gate_block(geom_ref[...], wgr_ref[...], wgs_ref[...])
    p = qd_ref[...] * kd_ref[...] * gate
    a = jnp.dot(p, mh_ref[...], preferred_element_type=jnp.float32)
    row = (pl.program_id(0) * BE +
           jax.lax.broadcasted_iota(jnp.int32, (BE, 16), 0))
    ex = jnp.where(row < E, jnp.exp(a), 0.0)
    exe = jnp.dot(ex, me_ref[...], preferred_element_type=jnp.float32)
    exe_ref[...] = exe
    msg_ref[...] = vd_ref[...] * exe


def _alpha_msg(geom, qd, kd, vd, wgr, wgs, mh16, me16):
    be = pl.BlockSpec((BE, 4), lambda i: (i, 0))
    bd = pl.BlockSpec((BE, D), lambda i: (i, 0))
    bw = pl.BlockSpec((D, D), lambda i: (0, 0))
    bs = pl.BlockSpec((16, D), lambda i: (0, 0))
    bm = pl.BlockSpec((D, 16), lambda i: (0, 0))
    bme = pl.BlockSpec((16, D), lambda i: (0, 0))
    return pl.pallas_call(
        _alpha_body,
        grid=(EP // BE,),
        in_specs=[be, bd, bd, bd, bw, bs, bm, bme],
        out_specs=[bd, bd],
        out_shape=[jax.ShapeDtypeStruct((EP, D), jnp.float32)] * 2,
    )(geom, qd, kd, vd, wgr, wgs, mh16, me16)


def _tail_body(x_ref, m0, m1, d0, d1, wo_, w1_, w2_, o_ref):
    agg = (m0[0] + m1[0]) / (d0[0] + d1[0] + 1e-9)
    t = x_ref[...] + jnp.dot(agg, wo_[...],
                             preferred_element_type=jnp.float32)
    h2 = _ln(t)
    u = jnp.dot(h2, w1_[...], preferred_element_type=jnp.float32)
    u = u * (1.0 / (1.0 + jnp.exp(-u)))
    o_ref[...] = t + jnp.dot(u, w2_[...], preferred_element_type=jnp.float32)


def _tail(x, om, od, wo, w1, w2):
    bx = pl.BlockSpec((BN, D), lambda i: (i, 0))
    bm0 = pl.BlockSpec((1, BN, D), lambda i: (0, i, 0))
    bm1 = pl.BlockSpec((1, BN, D), lambda i: (1, i, 0))
    bwo = pl.BlockSpec((D, D), lambda i: (0, 0))
    bw1 = pl.BlockSpec((D, 2 * D), lambda i: (0, 0))
    bw2 = pl.BlockSpec((2 * D, D), lambda i: (0, 0))
    return pl.pallas_call(
        _tail_body,
        grid=(NP_ // BN,),
        in_specs=[bx, bm0, bm1, bm0, bm1, bwo, bw1, bw2],
        out_specs=bx,
        out_shape=jax.ShapeDtypeStruct((NP_, D), jnp.float32),
    )(x, om, om, od, od, wo, w1, w2)


def _deg_body(geom_ref, wdeg_ref, o_ref):
    rbf = _rbf_from_len(geom_ref[:, 0:1])
    z = jnp.dot(rbf, wdeg_ref[...], preferred_element_type=jnp.float32)
    row = (pl.program_id(0) * BE +
           jax.lax.broadcasted_iota(jnp.int32, (BE, D), 0))
    o_ref[...] = jnp.where(row < E, z, 0.0)


def _deg(geom, wdeg):
    be = pl.BlockSpec((BE, 4), lambda i: (i, 0))
    bw = pl.BlockSpec((D, D), lambda i: (0, 0))
    bd = pl.BlockSpec((BE, D), lambda i: (i, 0))
    return pl.pallas_call(
        _deg_body,
        grid=(EP // BE,),
        in_specs=[be, bw],
        out_specs=bd,
        out_shape=jax.ShapeDtypeStruct((EP, D), jnp.float32),
    )(geom, wdeg)


def _head_body(x_ref, b_ref, w1_ref, w2_ref, o_ref):
    i = pl.program_id(0)

    @pl.when(i == 0)
    def _():
        o_ref[...] = jnp.zeros_like(o_ref)

    h = _ln(x_ref[...])
    u = jnp.dot(h, w1_ref[...], preferred_element_type=jnp.float32)
    u = u * (1.0 / (1.0 + jnp.exp(-u)))
    ne = jnp.dot(u, w2_ref[...], preferred_element_type=jnp.float32)
    ne0 = ne[:, 0:1]
    gids = jax.lax.broadcasted_iota(jnp.int32, (1, NG), 1)
    onehot = (b_ref[...] == gids).astype(jnp.float32)
    o_ref[...] += jnp.dot(ne0.T, onehot, preferred_element_type=jnp.float32)


def _head(x, batch2d, w1, w2p):
    bx = pl.BlockSpec((BN, D), lambda i: (i, 0))
    bb = pl.BlockSpec((BN, 1), lambda i: (i, 0))
    bw1 = pl.BlockSpec((D, D), lambda i: (0, 0))
    bw2 = pl.BlockSpec((D, 8), lambda i: (0, 0))
    bo = pl.BlockSpec((1, NG), lambda i: (0, 0))
    return pl.pallas_call(
        _head_body,
        grid=(NP_ // BN,),
        in_specs=[bx, bb, bw1, bw2],
        out_specs=bo,
        out_shape=jax.ShapeDtypeStruct((1, NG), jnp.float32),
    )(x, batch2d, w1, w2p)


def kernel(node_atom, pos, batch, edge_index, emb_table, W_deg, Wq, Wk, Wv,
           Wo, Wg_rbf, Wg_sh, W1, W2, head_w1, head_w2):
    src = edge_index[0].astype(jnp.int32)
    dst = edge_index[1].astype(jnp.int32)

    pvec = pos[src] - pos[dst]
    elen = jnp.sqrt(jnp.sum(pvec ** 2, axis=1) + 1e-12)
    unit = pvec / elen[:, None]
    geom = jnp.concatenate([elen[:, None], unit], axis=1)
    geom_p = jnp.zeros((EP, 4), jnp.float32).at[:E].set(geom)
    src_p = jnp.zeros((EP,), jnp.int32).at[:E].set(src)
    dst_p = jnp.full((EP,), N, jnp.int32).at[:E].set(dst)

    mh16 = (jax.lax.broadcasted_iota(jnp.int32, (D, 16), 0) // DH ==
            jax.lax.broadcasted_iota(jnp.int32, (D, 16), 1)).astype(
                jnp.float32) * (1.0 / math.sqrt(float(DH)))
    me16 = (jax.lax.broadcasted_iota(jnp.int32, (16, D), 1) // DH ==
            jax.lax.broadcasted_iota(jnp.int32, (16, D), 0)).astype(jnp.float32)

    wgs_pad = jnp.zeros((L, 16, D), jnp.float32).at[:, :9, :].set(Wg_sh)

    zrow = jnp.zeros((NP_, D), jnp.float32)

    z = _deg(geom_p, W_deg)
    degm = _sc_scatter(z, dst_p, zrow)
    deg = (degm[0, :N] + degm[1, :N]) / AVG_DEGREE
    x0 = emb_table[node_atom] + deg
    x = jnp.zeros((NP_, D), jnp.float32).at[:N].set(x0)

    for l in range(L):
        qn, kn, vn = _qkv(x, Wq[l], Wk[l], Wv[l])
        qd, kd, vd = _sc_gather3(qn, kn, vn, dst_p, src_p)
        msg, exe = _alpha_msg(geom_p, qd, kd, vd, Wg_rbf[l], wgs_pad[l],
                              mh16, me16)
        om = _sc_scatter(msg, dst_p, zrow)
        od = _sc_scatter(exe, dst_p, zrow)
        x = _tail(x, om, od, Wo[l], W1[l], W2[l])

    batch2d = jnp.full((NP_, 1), NG, jnp.int32).at[:N, 0].set(
        batch.astype(jnp.int32))
    w2p = jnp.zeros((D, 8), jnp.float32).at[:, 0:1].set(head_w2)
    energy = _head(x, batch2d, head_w1, w2p)
    return energy.reshape(NG) / AVG_NUM_NODES


# speedup vs baseline: 3.2654x; 1.5897x over previous
import functools
import math

import jax
import jax.numpy as jnp
from jax import lax
from jax.experimental import pallas as pl
from jax.experimental.pallas import tpu as pltpu
from jax.experimental.pallas import tpu_sc as plsc

N = 10000
E = 160000
D = 128
NB = 128
H = 4
DH = D // H
L = 6
NG = 512
MAX_RADIUS = 5.0
AVG_DEGREE = 15.57930850982666
AVG_NUM_NODES = 18.03065905448718

NP_ = 10240
BN = 1024
EP = 163840
BE = 4096
NWK = 32
EPW = EP // NWK
CH = 128
NCH = EPW // CH
NROW = NP_ // 16
NRC = NROW // CH


def _ln(x):
    m = jnp.mean(x, axis=-1, keepdims=True)
    v = jnp.mean((x - m) ** 2, axis=-1, keepdims=True)
    return (x - m) * jax.lax.rsqrt(v + 1e-5)


def _rbf_from_len(elen):
    width = MAX_RADIUS / NB
    centers = jax.lax.broadcasted_iota(jnp.int32, (1, NB), 1).astype(
        jnp.float32) * (MAX_RADIUS / (NB - 1))
    z = (elen - centers) * (1.0 / width)
    return jnp.exp(-(z * z))


def _gate_block(geom, wg_rbf, wg_sh):
    elen = geom[:, 0:1]
    ux = geom[:, 1:2]
    uy = geom[:, 2:3]
    uz = geom[:, 3:4]
    rbf = _rbf_from_len(elen)
    gate = jnp.dot(rbf, wg_rbf, preferred_element_type=jnp.float32)
    s3 = math.sqrt(3.0)
    s15 = math.sqrt(15.0)
    s5 = math.sqrt(5.0)
    coefs = [
        jnp.ones_like(ux),
        s3 * ux, s3 * uy, s3 * uz,
        s15 * ux * uy, s15 * uy * uz,
        0.5 * s5 * (3.0 * uz * uz - 1.0),
        s15 * ux * uz, 0.5 * s15 * (ux * ux - uy * uy),
    ]
    for j, c in enumerate(coefs):
        gate = gate + c * wg_sh[j:j + 1, :]
    return gate


_MESH = plsc.VectorSubcoreMesh(core_axis_name="c", subcore_axis_name="s")


@functools.partial(
    pl.kernel,
    mesh=_MESH,
    out_type=[
        jax.ShapeDtypeStruct((EP, D), jnp.float32),
        jax.ShapeDtypeStruct((EP, D), jnp.float32),
        jax.ShapeDtypeStruct((EP, D), jnp.float32),
    ],
    scratch_types=[
        pltpu.VMEM((2, CH), jnp.int32),
        pltpu.VMEM((2, CH), jnp.int32),
        pltpu.VMEM((2, CH, D), jnp.float32),
        pltpu.VMEM((2, CH, D), jnp.float32),
        pltpu.VMEM((2, CH, D), jnp.float32),
        pltpu.SemaphoreType.DMA,
        pltpu.SemaphoreType.DMA,
    ],
)
def _sc_gather3(qn, kn, vn, dst, src, qd, kd, vd,
                idxd_v, idxs_v, qb, kb, vb, sem0, sem1):
    cid = lax.axis_index("c")
    sid = lax.axis_index("s")
    base0 = (cid * 16 + sid) * EPW
    sems = (sem0, sem1)

    def start(it, b):
        base = base0 + it * CH
        pltpu.sync_copy(dst.at[pl.ds(base, CH)], idxd_v.at[b])
        pltpu.sync_copy(src.at[pl.ds(base, CH)], idxs_v.at[b])
        pltpu.async_copy(qn.at[idxd_v.at[b]], qb.at[b], sems[b])
        pltpu.async_copy(kn.at[idxs_v.at[b]], kb.at[b], sems[b])
        pltpu.async_copy(vn.at[idxs_v.at[b]], vb.at[b], sems[b])

    def finish(it, b):
        base = base0 + it * CH
        pltpu.make_async_copy(qn.at[idxd_v.at[b]], qb.at[b], sems[b]).wait()
        pltpu.make_async_copy(kn.at[idxs_v.at[b]], kb.at[b], sems[b]).wait()
        pltpu.make_async_copy(vn.at[idxs_v.at[b]], vb.at[b], sems[b]).wait()
        pltpu.sync_copy(qb.at[b], qd.at[pl.ds(base, CH)])
        pltpu.sync_copy(kb.at[b], kd.at[pl.ds(base, CH)])
        pltpu.sync_copy(vb.at[b], vd.at[pl.ds(base, CH)])

    start(0, 0)
    start(1, 1)

    def loop(g, _):
        for b in (0, 1):
            it = 2 * g + b
            finish(it, b)
            start(it + 2, b)
        return 0

    lax.fori_loop(0, NCH // 2 - 1, loop, 0)
    finish(NCH - 2, 0)
    finish(NCH - 1, 1)


@functools.partial(
    pl.kernel,
    mesh=_MESH,
    out_type=jax.ShapeDtypeStruct((2, NP_, D), jnp.float32),
    scratch_types=[
        pltpu.VMEM((2, CH), jnp.int32),
        pltpu.VMEM((2, CH, D), jnp.float32),
        pltpu.SemaphoreType.DMA,
        pltpu.SemaphoreType.DMA,
        pltpu.VMEM_SHARED((NP_, D), jnp.float32),
    ],
)
def _sc_scatter(rows, dst, zrow, om, idx_v, rows_v, sem0, sem1, accm):
    cid = lax.axis_index("c")
    sid = lax.axis_index("s")
    rbase = sid * NROW
    sems = (sem0, sem1)

    def zinit(i, _):
        rb = rbase + i * CH
        pltpu.sync_copy(zrow.at[pl.ds(rb, CH)], rows_v.at[0])
        pltpu.sync_copy(rows_v.at[0], accm.at[pl.ds(rb, CH)])
        return 0

    lax.fori_loop(0, NRC, zinit, 0)
    plsc.subcore_barrier()
    base0 = (cid * 16 + sid) * EPW

    def start(it, b):
        base = base0 + it * CH
        pltpu.async_copy(dst.at[pl.ds(base, CH)], idx_v.at[b], sems[b])
        pltpu.async_copy(rows.at[pl.ds(base, CH)], rows_v.at[b], sems[b])

    def finish(it, b):
        base = base0 + it * CH
        pltpu.make_async_copy(dst.at[pl.ds(base, CH)], idx_v.at[b],
                              sems[b]).wait()
        pltpu.make_async_copy(rows.at[pl.ds(base, CH)], rows_v.at[b],
                              sems[b]).wait()
        pltpu.sync_copy(rows_v.at[b], accm.at[idx_v.at[b]], add=True)

    start(0, 0)
    start(1, 1)

    def chunk(g, _):
        for b in (0, 1):
            it = 2 * g + b
            finish(it, b)
            start(it + 2, b)
        return 0

    lax.fori_loop(0, NCH // 2 - 1, chunk, 0)
    finish(NCH - 2, 0)
    finish(NCH - 1, 1)
    plsc.subcore_barrier()

    def wback(i, _):
        rb = rbase + i * CH
        pltpu.sync_copy(accm.at[pl.ds(rb, CH)], rows_v.at[0])
        pltpu.sync_copy(rows_v.at[0], om.at[cid, pl.ds(rb, CH)])
        return 0

    lax.fori_loop(0, NRC, wback, 0)


def _qkv_body(x_ref, wq_ref, wk_ref, wv_ref, q_ref, k_ref, v_ref):
    h = _ln(x_ref[...])
    q_ref[...] = jnp.dot(h, wq_ref[...], preferred_element_type=jnp.float32)
    k_ref[...] = jnp.dot(h, wk_ref[...], preferred_element_type=jnp.float32)
    v_ref[...] = jnp.dot(h, wv_ref[...], preferred_element_type=jnp.float32)


def _qkv(x, wq, wk, wv):
    bx = pl.BlockSpec((BN, D), lambda i: (i, 0))
    bw = pl.BlockSpec((D, D), lambda i: (0, 0))
    return pl.pallas_call(
        _qkv_body,
        grid=(NP_ // BN,),
        in_specs=[bx, bw, bw, bw],
        out_specs=[bx, bx, bx],
        out_shape=[jax.ShapeDtypeStruct((NP_, D), jnp.float32)] * 3,
    )(x, wq, wk, wv)


def _alpha_body(geom_ref, qd_ref, kd_ref, vd_ref, wgr_ref, wgs_ref, mh_ref,
                me_ref, msg_ref, exe_ref):
    gate = _gate_block(geom_ref[...], wgr_ref[...], wgs_ref[...])
    p = qd_ref[...] * kd_ref[...] * gate
    a = jnp.dot(p, mh_ref[...], preferred_element_type=jnp.float32)
    row = (pl.program_id(0) * BE +
           jax.lax.broadcasted_iota(jnp.int32, (BE, 16), 0))
    ex = jnp.where(row < E, jnp.exp(a), 0.0)
    exe = jnp.dot(ex, me_ref[...], preferred_element_type=jnp.float32)
    exe_ref[...] = exe
    msg_ref[...] = vd_ref[...] * exe


def _alpha_msg(geom, qd, kd, vd, wgr, wgs, mh16, me16):
    be = pl.BlockSpec((BE, 4), lambda i: (i, 0))
    bd = pl.BlockSpec((BE, D), lambda i: (i, 0))
    bw = pl.BlockSpec((D, D), lambda i: (0, 0))
    bs = pl.BlockSpec((16, D), lambda i: (0, 0))
    bm = pl.BlockSpec((D, 16), lambda i: (0, 0))
    bme = pl.BlockSpec((16, D), lambda i: (0, 0))
    return pl.pallas_call(
        _alpha_body,
        grid=(EP // BE,),
        in_specs=[be, bd, bd, bd, bw, bs, bm, bme],
        out_specs=[bd, bd],
        out_shape=[jax.ShapeDtypeStruct((EP, D), jnp.float32)] * 2,
    )(geom, qd, kd, vd, wgr, wgs, mh16, me16)


def _tail_body(x_ref, m0, m1, d0, d1, wo_, w1_, w2_, o_ref):
    agg = (m0[0] + m1[0]) / (d0[0] + d1[0] + 1e-9)
    t = x_ref[...] + jnp.dot(agg, wo_[...],
                             preferred_element_type=jnp.float32)
    h2 = _ln(t)
    u = jnp.dot(h2, w1_[...], preferred_element_type=jnp.float32)
    u = u * (1.0 / (1.0 + jnp.exp(-u)))
    o_ref[...] = t + jnp.dot(u, w2_[...], preferred_element_type=jnp.float32)


def _tail(x, om, od, wo, w1, w2):
    bx = pl.BlockSpec((BN, D), lambda i: (i, 0))
    bm0 = pl.BlockSpec((1, BN, D), lambda i: (0, i, 0))
    bm1 = pl.BlockSpec((1, BN, D), lambda i: (1, i, 0))
    bwo = pl.BlockSpec((D, D), lambda i: (0, 0))
    bw1 = pl.BlockSpec((D, 2 * D), lambda i: (0, 0))
    bw2 = pl.BlockSpec((2 * D, D), lambda i: (0, 0))
    return pl.pallas_call(
        _tail_body,
        grid=(NP_ // BN,),
        in_specs=[bx, bm0, bm1, bm0, bm1, bwo, bw1, bw2],
        out_specs=bx,
        out_shape=jax.ShapeDtypeStruct((NP_, D), jnp.float32),
    )(x, om, om, od, od, wo, w1, w2)


def _deg_body(geom_ref, wdeg_ref, o_ref):
    rbf = _rbf_from_len(geom_ref[:, 0:1])
    z = jnp.dot(rbf, wdeg_ref[...], preferred_element_type=jnp.float32)
    row = (pl.program_id(0) * BE +
           jax.lax.broadcasted_iota(jnp.int32, (BE, D), 0))
    o_ref[...] = jnp.where(row < E, z, 0.0)


def _deg(geom, wdeg):
    be = pl.BlockSpec((BE, 4), lambda i: (i, 0))
    bw = pl.BlockSpec((D, D), lambda i: (0, 0))
    bd = pl.BlockSpec((BE, D), lambda i: (i, 0))
    return pl.pallas_call(
        _deg_body,
        grid=(EP // BE,),
        in_specs=[be, bw],
        out_specs=bd,
        out_shape=jax.ShapeDtypeStruct((EP, D), jnp.float32),
    )(geom, wdeg)


def _head_body(x_ref, b_ref, w1_ref, w2_ref, o_ref):
    i = pl.program_id(0)

    @pl.when(i == 0)
    def _():
        o_ref[...] = jnp.zeros_like(o_ref)

    h = _ln(x_ref[...])
    u = jnp.dot(h, w1_ref[...], preferred_element_type=jnp.float32)
    u = u * (1.0 / (1.0 + jnp.exp(-u)))
    ne = jnp.dot(u, w2_ref[...], preferred_element_type=jnp.float32)
    ne0 = ne[:, 0:1]
    gids = jax.lax.broadcasted_iota(jnp.int32, (1, NG), 1)
    onehot = (b_ref[...] == gids).astype(jnp.float32)
    o_ref[...] += jnp.dot(ne0.T, onehot, preferred_element_type=jnp.float32)


def _head(x, batch2d, w1, w2p):
    bx = pl.BlockSpec((BN, D), lambda i: (i, 0))
    bb = pl.BlockSpec((BN, 1), lambda i: (i, 0))
    bw1 = pl.BlockSpec((D, D), lambda i: (0, 0))
    bw2 = pl.BlockSpec((D, 8), lambda i: (0, 0))
    bo = pl.BlockSpec((1, NG), lambda i: (0, 0))
    return pl.pallas_call(
        _head_body,
        grid=(NP_ // BN,),
        in_specs=[bx, bb, bw1, bw2],
        out_specs=bo,
        out_shape=jax.ShapeDtypeStruct((1, NG), jnp.float32),
    )(x, batch2d, w1, w2p)


def kernel(node_atom, pos, batch, edge_index, emb_table, W_deg, Wq, Wk, Wv,
           Wo, Wg_rbf, Wg_sh, W1, W2, head_w1, head_w2):
    src = edge_index[0].astype(jnp.int32)
    dst = edge_index[1].astype(jnp.int32)

    pvec = pos[src] - pos[dst]
    elen = jnp.sqrt(jnp.sum(pvec ** 2, axis=1) + 1e-12)
    unit = pvec / elen[:, None]
    geom = jnp.concatenate([elen[:, None], unit], axis=1)
    geom_p = jnp.zeros((EP, 4), jnp.float32).at[:E].set(geom)
    src_p = jnp.zeros((EP,), jnp.int32).at[:E].set(src)
    dst_p = jnp.full((EP,), N, jnp.int32).at[:E].set(dst)

    mh16 = (jax.lax.broadcasted_iota(jnp.int32, (D, 16), 0) // DH ==
            jax.lax.broadcasted_iota(jnp.int32, (D, 16), 1)).astype(
                jnp.float32) * (1.0 / math.sqrt(float(DH)))
    me16 = (jax.lax.broadcasted_iota(jnp.int32, (16, D), 1) // DH ==
            jax.lax.broadcasted_iota(jnp.int32, (16, D), 0)).astype(jnp.float32)

    wgs_pad = jnp.zeros((L, 16, D), jnp.float32).at[:, :9, :].set(Wg_sh)

    zrow = jnp.zeros((NP_, D), jnp.float32)

    z = _deg(geom_p, W_deg)
    degm = _sc_scatter(z, dst_p, zrow)
    deg = (degm[0, :N] + degm[1, :N]) / AVG_DEGREE
    x0 = emb_table[node_atom] + deg
    x = jnp.zeros((NP_, D), jnp.float32).at[:N].set(x0)

    for l in range(L):
        qn, kn, vn = _qkv(x, Wq[l], Wk[l], Wv[l])
        qd, kd, vd = _sc_gather3(qn, kn, vn, dst_p, src_p)
        msg, exe = _alpha_msg(geom_p, qd, kd, vd, Wg_rbf[l], wgs_pad[l],
                              mh16, me16)
        om = _sc_scatter(msg, dst_p, zrow)
        od = _sc_scatter(exe, dst_p, zrow)
        x = _tail(x, om, od, Wo[l], W1[l], W2[l])

    batch2d = jnp.full((NP_, 1), NG, jnp.int32).at[:N, 0].set(
        batch.astype(jnp.int32))
    w2p = jnp.zeros((D, 8), jnp.float32).at[:, 0:1].set(head_w2)
    energy = _head(x, batch2d, head_w1, w2p)
    return energy.reshape(NG) / AVG_NUM_NODES


# speedup vs baseline: 3.4953x; 1.0704x over previous
import functools
import math

import jax
import jax.numpy as jnp
from jax import lax
from jax.experimental import pallas as pl
from jax.experimental.pallas import tpu as pltpu
from jax.experimental.pallas import tpu_sc as plsc

N = 10000
E = 160000
D = 128
NB = 128
H = 4
DH = D // H
L = 6
NG = 512
MAX_RADIUS = 5.0
AVG_DEGREE = 15.57930850982666
AVG_NUM_NODES = 18.03065905448718

NP_ = 10240
BN = 1024
EP = 163840
BE = 4096
NWK = 32
EPW = EP // NWK
CH = 128
NCH = EPW // CH
NROW = NP_ // 16
NRC = NROW // CH


def _ln(x):
    m = jnp.mean(x, axis=-1, keepdims=True)
    v = jnp.mean((x - m) ** 2, axis=-1, keepdims=True)
    return (x - m) * jax.lax.rsqrt(v + 1e-5)


def _rbf_from_len(elen):
    width = MAX_RADIUS / NB
    centers = jax.lax.broadcasted_iota(jnp.int32, (1, NB), 1).astype(
        jnp.float32) * (MAX_RADIUS / (NB - 1))
    z = (elen - centers) * (1.0 / width)
    return jnp.exp(-(z * z))


def _gate_block(geom, wg_rbf, wg_sh):
    elen = geom[:, 0:1]
    ux = geom[:, 1:2]
    uy = geom[:, 2:3]
    uz = geom[:, 3:4]
    rbf = _rbf_from_len(elen)
    gate = jnp.dot(rbf, wg_rbf, preferred_element_type=jnp.float32)
    s3 = math.sqrt(3.0)
    s15 = math.sqrt(15.0)
    s5 = math.sqrt(5.0)
    coefs = [
        jnp.ones_like(ux),
        s3 * ux, s3 * uy, s3 * uz,
        s15 * ux * uy, s15 * uy * uz,
        0.5 * s5 * (3.0 * uz * uz - 1.0),
        s15 * ux * uz, 0.5 * s15 * (ux * ux - uy * uy),
    ]
    for j, c in enumerate(coefs):
        gate = gate + c * wg_sh[j:j + 1, :]
    return gate


_MESH = plsc.VectorSubcoreMesh(core_axis_name="c", subcore_axis_name="s")


@functools.partial(
    pl.kernel,
    mesh=_MESH,
    out_type=[
        jax.ShapeDtypeStruct((EP, D), jnp.float32),
        jax.ShapeDtypeStruct((EP, D), jnp.float32),
    ],
    scratch_types=[
        pltpu.VMEM((2, CH), jnp.int32),
        pltpu.VMEM((2, CH), jnp.int32),
        pltpu.VMEM((2, CH, D), jnp.float32),
        pltpu.VMEM((2, CH, D), jnp.float32),
        pltpu.SemaphoreType.DMA,
        pltpu.SemaphoreType.DMA,
    ],
)
def _sc_gather2(qn, kv, dst, src, qd, kvd,
                idxd_v, idxs_v, qb, kb, sem0, sem1):
    cid = lax.axis_index("c")
    sid = lax.axis_index("s")
    base0 = (cid * 16 + sid) * EPW
    sems = (sem0, sem1)

    def start(it, b):
        base = base0 + it * CH
        pltpu.sync_copy(dst.at[pl.ds(base, CH)], idxd_v.at[b])
        pltpu.sync_copy(src.at[pl.ds(base, CH)], idxs_v.at[b])
        pltpu.async_copy(qn.at[idxd_v.at[b]], qb.at[b], sems[b])
        pltpu.async_copy(kv.at[idxs_v.at[b]], kb.at[b], sems[b])

    def finish(it, b):
        base = base0 + it * CH
        pltpu.make_async_copy(qn.at[idxd_v.at[b]], qb.at[b], sems[b]).wait()
        pltpu.make_async_copy(kv.at[idxs_v.at[b]], kb.at[b], sems[b]).wait()
        pltpu.sync_copy(qb.at[b], qd.at[pl.ds(base, CH)])
        pltpu.sync_copy(kb.at[b], kvd.at[pl.ds(base, CH)])

    start(0, 0)
    start(1, 1)

    def loop(g, _):
        for b in (0, 1):
            it = 2 * g + b
            finish(it, b)
            start(it + 2, b)
        return 0

    lax.fori_loop(0, NCH // 2 - 1, loop, 0)
    finish(NCH - 2, 0)
    finish(NCH - 1, 1)


@functools.partial(
    pl.kernel,
    mesh=_MESH,
    out_type=jax.ShapeDtypeStruct((2, NP_, D), jnp.float32),
    scratch_types=[
        pltpu.VMEM((2, CH), jnp.int32),
        pltpu.VMEM((2, CH, D), jnp.float32),
        pltpu.SemaphoreType.DMA,
        pltpu.SemaphoreType.DMA,
        pltpu.VMEM_SHARED((NP_, D), jnp.float32),
    ],
)
def _sc_scatter(rows, dst, zrow, om, idx_v, rows_v, sem0, sem1, accm):
    cid = lax.axis_index("c")
    sid = lax.axis_index("s")
    rbase = sid * NROW
    sems = (sem0, sem1)

    def zinit(i, _):
        rb = rbase + i * CH
        pltpu.sync_copy(zrow.at[pl.ds(rb, CH)], rows_v.at[0])
        pltpu.sync_copy(rows_v.at[0], accm.at[pl.ds(rb, CH)])
        return 0

    lax.fori_loop(0, NRC, zinit, 0)
    plsc.subcore_barrier()
    base0 = (cid * 16 + sid) * EPW

    def start(it, b):
        base = base0 + it * CH
        pltpu.async_copy(dst.at[pl.ds(base, CH)], idx_v.at[b], sems[b])
        pltpu.async_copy(rows.at[pl.ds(base, CH)], rows_v.at[b], sems[b])

    def finish(it, b):
        base = base0 + it * CH
        pltpu.make_async_copy(dst.at[pl.ds(base, CH)], idx_v.at[b],
                              sems[b]).wait()
        pltpu.make_async_copy(rows.at[pl.ds(base, CH)], rows_v.at[b],
                              sems[b]).wait()
        pltpu.sync_copy(rows_v.at[b], accm.at[idx_v.at[b]], add=True)

    start(0, 0)
    start(1, 1)

    def chunk(g, _):
        for b in (0, 1):
            it = 2 * g + b
            finish(it, b)
            start(it + 2, b)
        return 0

    lax.fori_loop(0, NCH // 2 - 1, chunk, 0)
    finish(NCH - 2, 0)
    finish(NCH - 1, 1)
    plsc.subcore_barrier()

    def wback(i, _):
        rb = rbase + i * CH
        pltpu.sync_copy(accm.at[pl.ds(rb, CH)], rows_v.at[0])
        pltpu.sync_copy(rows_v.at[0], om.at[cid, pl.ds(rb, CH)])
        return 0

    lax.fori_loop(0, NRC, wback, 0)


def _qkv_body(x_ref, wq_ref, wk_ref, wv_ref, q_ref, kv_ref):
    h = _ln(x_ref[...])
    q_ref[...] = jnp.dot(h, wq_ref[...], preferred_element_type=jnp.float32)
    k = jnp.dot(h, wk_ref[...], preferred_element_type=jnp.float32)
    v = jnp.dot(h, wv_ref[...], preferred_element_type=jnp.float32)
    k32 = jax.lax.bitcast_convert_type(k, jnp.uint32)
    v32 = jax.lax.bitcast_convert_type(v, jnp.uint32)
    hi = (k32 + 0x8000) & jnp.uint32(0xFFFF0000)
    lo = (v32 + 0x8000) >> 16
    kv_ref[...] = jax.lax.bitcast_convert_type(hi | lo, jnp.float32)


def _qkv(x, wq, wk, wv):
    bx = pl.BlockSpec((BN, D), lambda i: (i, 0))
    bw = pl.BlockSpec((D, D), lambda i: (0, 0))
    return pl.pallas_call(
        _qkv_body,
        grid=(NP_ // BN,),
        in_specs=[bx, bw, bw, bw],
        out_specs=[bx, bx],
        out_shape=[jax.ShapeDtypeStruct((NP_, D), jnp.float32)] * 2,
    )(x, wq, wk, wv)


def _alpha_body(geom_ref, qd_ref, kvd_ref, wgr_ref, wgs_ref, mh_ref,
                me_ref, msg_ref, exe_ref):
    gate = _gate_block(geom_ref[...], wgr_ref[...], wgs_ref[...])
    kv32 = jax.lax.bitcast_convert_type(kvd_ref[...], jnp.uint32)
    kd = jax.lax.bitcast_convert_type(kv32 & jnp.uint32(0xFFFF0000),
                                      jnp.float32)
    vd = jax.lax.bitcast_convert_type(kv32 << 16, jnp.float32)
    p = qd_ref[...] * kd * gate
    a = jnp.dot(p, mh_ref[...], preferred_element_type=jnp.float32)
    row = (pl.program_id(0) * BE +
           jax.lax.broadcasted_iota(jnp.int32, (BE, 16), 0))
    ex = jnp.where(row < E, jnp.exp(a), 0.0)
    exe = jnp.dot(ex, me_ref[...], preferred_element_type=jnp.float32)
    exe_ref[...] = exe
    msg_ref[...] = vd * exe


def _alpha_msg(geom, qd, kvd, wgr, wgs, mh16, me16):
    be = pl.BlockSpec((BE, 4), lambda i: (i, 0))
    bd = pl.BlockSpec((BE, D), lambda i: (i, 0))
    bw = pl.BlockSpec((D, D), lambda i: (0, 0))
    bs = pl.BlockSpec((16, D), lambda i: (0, 0))
    bm = pl.BlockSpec((D, 16), lambda i: (0, 0))
    bme = pl.BlockSpec((16, D), lambda i: (0, 0))
    return pl.pallas_call(
        _alpha_body,
        grid=(EP // BE,),
        in_specs=[be, bd, bd, bw, bs, bm, bme],
        out_specs=[bd, bd],
        out_shape=[jax.ShapeDtypeStruct((EP, D), jnp.float32)] * 2,
    )(geom, qd, kvd, wgr, wgs, mh16, me16)


def _tail_body(x_ref, m0, m1, d0, d1, wo_, w1_, w2_, o_ref):
    agg = (m0[0] + m1[0]) / (d0[0] + d1[0] + 1e-9)
    t = x_ref[...] + jnp.dot(agg, wo_[...],
                             preferred_element_type=jnp.float32)
    h2 = _ln(t)
    u = jnp.dot(h2, w1_[...], preferred_element_type=jnp.float32)
    u = u * (1.0 / (1.0 + jnp.exp(-u)))
    o_ref[...] = t + jnp.dot(u, w2_[...], preferred_element_type=jnp.float32)


def _tail(x, om, od, wo, w1, w2):
    bx = pl.BlockSpec((BN, D), lambda i: (i, 0))
    bm0 = pl.BlockSpec((1, BN, D), lambda i: (0, i, 0))
    bm1 = pl.BlockSpec((1, BN, D), lambda i: (1, i, 0))
    bwo = pl.BlockSpec((D, D), lambda i: (0, 0))
    bw1 = pl.BlockSpec((D, 2 * D), lambda i: (0, 0))
    bw2 = pl.BlockSpec((2 * D, D), lambda i: (0, 0))
    return pl.pallas_call(
        _tail_body,
        grid=(NP_ // BN,),
        in_specs=[bx, bm0, bm1, bm0, bm1, bwo, bw1, bw2],
        out_specs=bx,
        out_shape=jax.ShapeDtypeStruct((NP_, D), jnp.float32),
    )(x, om, om, od, od, wo, w1, w2)


def _deg_body(geom_ref, wdeg_ref, o_ref):
    rbf = _rbf_from_len(geom_ref[:, 0:1])
    z = jnp.dot(rbf, wdeg_ref[...], preferred_element_type=jnp.float32)
    row = (pl.program_id(0) * BE +
           jax.lax.broadcasted_iota(jnp.int32, (BE, D), 0))
    o_ref[...] = jnp.where(row < E, z, 0.0)


def _deg(geom, wdeg):
    be = pl.BlockSpec((BE, 4), lambda i: (i, 0))
    bw = pl.BlockSpec((D, D), lambda i: (0, 0))
    bd = pl.BlockSpec((BE, D), lambda i: (i, 0))
    return pl.pallas_call(
        _deg_body,
        grid=(EP // BE,),
        in_specs=[be, bw],
        out_specs=bd,
        out_shape=jax.ShapeDtypeStruct((EP, D), jnp.float32),
    )(geom, wdeg)


def _head_body(x_ref, b_ref, w1_ref, w2_ref, o_ref):
    i = pl.program_id(0)

    @pl.when(i == 0)
    def _():
        o_ref[...] = jnp.zeros_like(o_ref)

    h = _ln(x_ref[...])
    u = jnp.dot(h, w1_ref[...], preferred_element_type=jnp.float32)
    u = u * (1.0 / (1.0 + jnp.exp(-u)))
    ne = jnp.dot(u, w2_ref[...], preferred_element_type=jnp.float32)
    ne0 = ne[:, 0:1]
    gids = jax.lax.broadcasted_iota(jnp.int32, (1, NG), 1)
    onehot = (b_ref[...] == gids).astype(jnp.float32)
    o_ref[...] += jnp.dot(ne0.T, onehot, preferred_element_type=jnp.float32)


def _head(x, batch2d, w1, w2p):
    bx = pl.BlockSpec((BN, D), lambda i: (i, 0))
    bb = pl.BlockSpec((BN, 1), lambda i: (i, 0))
    bw1 = pl.BlockSpec((D, D), lambda i: (0, 0))
    bw2 = pl.BlockSpec((D, 8), lambda i: (0, 0))
    bo = pl.BlockSpec((1, NG), lambda i: (0, 0))
    return pl.pallas_call(
        _head_body,
        grid=(NP_ // BN,),
        in_specs=[bx, bb, bw1, bw2],
        out_specs=bo,
        out_shape=jax.ShapeDtypeStruct((1, NG), jnp.float32),
    )(x, batch2d, w1, w2p)


def kernel(node_atom, pos, batch, edge_index, emb_table, W_deg, Wq, Wk, Wv,
           Wo, Wg_rbf, Wg_sh, W1, W2, head_w1, head_w2):
    src = edge_index[0].astype(jnp.int32)
    dst = edge_index[1].astype(jnp.int32)

    pvec = pos[src] - pos[dst]
    elen = jnp.sqrt(jnp.sum(pvec ** 2, axis=1) + 1e-12)
    unit = pvec / elen[:, None]
    geom = jnp.concatenate([elen[:, None], unit], axis=1)
    geom_p = jnp.zeros((EP, 4), jnp.float32).at[:E].set(geom)
    src_p = jnp.zeros((EP,), jnp.int32).at[:E].set(src)
    dst_p = jnp.full((EP,), N, jnp.int32).at[:E].set(dst)

    mh16 = (jax.lax.broadcasted_iota(jnp.int32, (D, 16), 0) // DH ==
            jax.lax.broadcasted_iota(jnp.int32, (D, 16), 1)).astype(
                jnp.float32) * (1.0 / math.sqrt(float(DH)))
    me16 = (jax.lax.broadcasted_iota(jnp.int32, (16, D), 1) // DH ==
            jax.lax.broadcasted_iota(jnp.int32, (16, D), 0)).astype(jnp.float32)

    wgs_pad = jnp.zeros((L, 16, D), jnp.float32).at[:, :9, :].set(Wg_sh)

    zrow = jnp.zeros((NP_, D), jnp.float32)

    z = _deg(geom_p, W_deg)
    degm = _sc_scatter(z, dst_p, zrow)
    deg = (degm[0, :N] + degm[1, :N]) / AVG_DEGREE
    x0 = emb_table[node_atom] + deg
    x = jnp.zeros((NP_, D), jnp.float32).at[:N].set(x0)

    for l in range(L):
        qn, kv = _qkv(x, Wq[l], Wk[l], Wv[l])
        qd, kvd = _sc_gather2(qn, kv, dst_p, src_p)
        msg, exe = _alpha_msg(geom_p, qd, kvd, Wg_rbf[l], wgs_pad[l],
                              mh16, me16)
        om = _sc_scatter(msg, dst_p, zrow)
        od = _sc_scatter(exe, dst_p, zrow)
        x = _tail(x, om, od, Wo[l], W1[l], W2[l])

    batch2d = jnp.full((NP_, 1), NG, jnp.int32).at[:N, 0].set(
        batch.astype(jnp.int32))
    w2p = jnp.zeros((D, 8), jnp.float32).at[:, 0:1].set(head_w2)
    energy = _head(x, batch2d, head_w1, w2p)
    return energy.reshape(NG) / AVG_NUM_NODES
